# baseline (speedup 1.0000x reference)
import functools

import jax
import jax.numpy as jnp
from jax import lax
from jax.experimental import pallas as pl
from jax.experimental.pallas import tpu as pltpu
from jax.experimental.pallas import tpu_sc as plsc

_CL = 1024
_NW = 32


def _sc_body(ct_hbm, table_hbm, x_hbm, out_hbm, ct_v, ttab_v, buf_v):
    c = lax.axis_index("c")
    s = lax.axis_index("s")
    w = s * 2 + c

    pltpu.sync_copy(ct_hbm, ct_v)
    pltpu.sync_copy(table_hbm, ttab_v)
    ct16 = plsc.load_gather(ct_v, [jnp.zeros((16,), jnp.int32)])

    def add_chunk(base, cols):
        pltpu.sync_copy(x_hbm.at[:, pl.ds(base, cols)], buf_v.at[:, pl.ds(0, cols)])
        for j in range(64):
            vc = plsc.load_gather(ttab_v, [ct16, jnp.full((16,), j, jnp.int32)])

            def inner(i, _):
                plsc.addupdate(buf_v.at[j, pl.ds(i * 16, 16)], vc)
                return 0

            lax.fori_loop(0, cols // 16, inner, 0)
        pltpu.sync_copy(buf_v.at[:, pl.ds(0, cols)], out_hbm.at[:, pl.ds(base, cols)])

    nchunks = 6 + jnp.where(w < 3, 1, 0)

    def chunk_loop(k, _):
        add_chunk((w + _NW * k) * _CL, _CL)
        return 0

    lax.fori_loop(0, nchunks, chunk_loop, 0)

    @pl.when(w == 3)
    def _():
        add_chunk(195 * _CL, 256)


def kernel(x, cell_type_id, table):
    n, d = x.shape
    xt = x.T
    ct = cell_type_id.astype(jnp.int32)

    mesh = plsc.VectorSubcoreMesh(core_axis_name="c", subcore_axis_name="s")
    sc_add = functools.partial(
        pl.kernel,
        out_type=jax.ShapeDtypeStruct((d, n), jnp.float32),
        mesh=mesh,
        scratch_types=[
            pltpu.VMEM((1,), jnp.int32),
            pltpu.VMEM(table.shape, jnp.float32),
            pltpu.VMEM((d, _CL), jnp.float32),
        ],
        compiler_params=pltpu.CompilerParams(
            needs_layout_passes=False,
            use_tc_tiling_on_sc=True,
        ),
    )(_sc_body)
    outt = sc_add(ct, table, xt)
    tail = (x[n - 64 :, :] + jnp.take(table, ct[0:1], axis=0)).T
    outt = lax.dynamic_update_slice(outt, tail, (0, n - 64))
    return outt.T


# speedup vs baseline: 3.2315x; 3.2315x over previous
import functools

import jax
import jax.numpy as jnp
from jax import lax
from jax.experimental import pallas as pl
from jax.experimental.pallas import tpu as pltpu
from jax.experimental.pallas import tpu_sc as plsc

_BLOCK_COLS = 49152


def _sc_lookup_body(ct_hbm, table_hbm, patt_hbm, ct_v, ttab_v, patt_v):
    c = lax.axis_index("c")
    s = lax.axis_index("s")

    @pl.when(jnp.logical_and(c == 0, s == 0))
    def _():
        pltpu.sync_copy(ct_hbm, ct_v)
        pltpu.sync_copy(table_hbm, ttab_v)
        ct16 = plsc.load_gather(ct_v, [jnp.zeros((16,), jnp.int32)])
        for j in range(64):
            v = plsc.load_gather(ttab_v, [ct16, jnp.full((16,), j, jnp.int32)])
            for l in range(8):
                patt_v[j, pl.ds(16 * l, 16)] = v
        pltpu.sync_copy(patt_v, patt_hbm)


def _tc_body(id_ref, tt_ref, x_ref, o_ref):
    ct = id_ref[0]
    tt = tt_ref[...]
    lane = jax.lax.broadcasted_iota(jnp.int32, tt.shape, 1)
    col = jnp.sum(jnp.where(lane == ct, tt, 0.0), axis=1, keepdims=True)
    o_ref[...] = x_ref[...] + col


def kernel(x, cell_type_id, table):
    n, d = x.shape
    xt = x.T
    tt = table.T
    ct = cell_type_id.astype(jnp.int32)

    mesh = plsc.VectorSubcoreMesh(core_axis_name="c", subcore_axis_name="s")
    sc_lookup = functools.partial(
        pl.kernel,
        out_type=jax.ShapeDtypeStruct((d, 128), jnp.float32),
        mesh=mesh,
        scratch_types=[
            pltpu.VMEM((1,), jnp.int32),
            pltpu.VMEM(table.shape, jnp.float32),
            pltpu.VMEM((d, 128), jnp.float32),
        ],
        compiler_params=pltpu.CompilerParams(
            needs_layout_passes=False, skip_device_barrier=True
        ),
    )(_sc_lookup_body)
    patt = sc_lookup(ct, table)

    grid = pl.cdiv(n, _BLOCK_COLS)
    outt = pl.pallas_call(
        _tc_body,
        grid=(grid,),
        in_specs=[
            pl.BlockSpec(memory_space=pltpu.SMEM),
            pl.BlockSpec((d, tt.shape[1]), lambda i: (0, 0)),
            pl.BlockSpec((d, _BLOCK_COLS), lambda i: (0, i)),
        ],
        out_specs=pl.BlockSpec((d, _BLOCK_COLS), lambda i: (0, i)),
        out_shape=jax.ShapeDtypeStruct((d, n), jnp.float32),
        compiler_params=pltpu.CompilerParams(
            dimension_semantics=("parallel",),
        ),
    )(ct, tt, xt)

    tail = xt[:, n - 128 :] + patt[:, 0:1]
    outt = lax.dynamic_update_slice(outt, tail, (0, n - 128))
    return outt.T
